# baseline (speedup 1.0000x reference)
import functools

import jax
import jax.numpy as jnp
from jax import lax
from jax.experimental import pallas as pl
from jax.experimental.pallas import tpu as pltpu
from jax.experimental.pallas import tpu_sc as plsc

_N = 25000
_E = 150000
_D = 128
_COEF = 0.1
_BN_EPS = 1e-5

_NC = 2
_NT = 16
_CH = 112


@functools.lru_cache(maxsize=None)
def _build_sc_segsum(n, e, kind):
    nch = e // _CH
    assert nch * _CH == e
    q = ((n + _NC * _NT - 1) // (_NC * _NT) + 7) // 8 * 8
    split = _NT * q
    trash = split
    acc_rows = split + 4 * _NT
    last = n - split - (_NT - 1) * q
    assert 0 < last <= q and split <= n and e % 8 == 0

    mesh = plsc.VectorSubcoreMesh(core_axis_name="c", subcore_axis_name="s")
    f32 = jnp.float32
    osd = jax.ShapeDtypeStruct((n, _D), f32)
    nout = 2 if kind == "ab" else 1

    @functools.partial(
        pl.kernel,
        out_type=(osd,) * nout,
        mesh=mesh,
        scratch_types=[
            pltpu.VMEM_SHARED((acc_rows, _D), f32),
            [pltpu.VMEM((2 * _CH,), jnp.int32)] * 2,
            [pltpu.VMEM((_CH,), jnp.int32)] * 2,
            [pltpu.VMEM((_CH, _D), f32)] * 2,
            pltpu.SemaphoreType.DMA,
            pltpu.SemaphoreType.DMA,
            pltpu.SemaphoreType.DMA,
        ],
    )
    def seg(*refs):
        x0, x1, p1, p2 = refs[:4]
        outs = refs[4:4 + nout]
        (acc, idx_v, dl_v, rows_v,
         sem_i, sem_g, sem_s) = refs[4 + nout:]
        c = lax.axis_index("c")
        s = lax.axis_index("s")
        lo = c * split
        hi = jnp.where(c == 0, split, n)
        base = s * q

        def _scan_edges(xt, pkt):
            nk = (nch - 1 - s) // _NT + 1

            def _src(b):
                return idx_v[b].at[pl.ds(0, _CH)]

            def _issue_idx(k, b):
                off = (s + k * _NT) * 2 * _CH
                pltpu.async_copy(pkt.at[pl.ds(off, 2 * _CH)], idx_v[b],
                                 sem_i)

            def _wait_idx(k, b):
                off = (s + k * _NT) * 2 * _CH
                pltpu.make_async_copy(pkt.at[pl.ds(off, 2 * _CH)],
                                      idx_v[b], sem_i).wait()

            tr = trash + s * 4 + (lax.iota(jnp.int32, 16) & 3)

            def _chunk(k, b):
                @pl.when(k > 0)
                def _():
                    pltpu.make_async_copy(xt.at[_src(1 - b)],
                                          rows_v[1 - b], sem_g).wait()

                @pl.when(k > 1)
                def _():
                    pltpu.make_async_copy(rows_v[b], acc.at[dl_v[b]],
                                          sem_s).wait()

                @pl.when(k + 1 < nk)
                def _():
                    _issue_idx(k + 1, 1 - b)

                _wait_idx(k, b)
                for j in range(_CH // 16):
                    d = idx_v[b][pl.ds(_CH + j * 16, 16)]
                    ok = (d >= lo) & (d < hi)
                    dl_v[b][pl.ds(j * 16, 16)] = jnp.where(ok, d - lo, tr)

                pltpu.async_copy(xt.at[_src(b)], rows_v[b], sem_g)

                @pl.when(k > 0)
                def _():
                    pltpu.async_copy(rows_v[1 - b], acc.at[dl_v[1 - b]],
                                     sem_s, add=True)

            _issue_idx(0, 0)

            def body(p, _):
                _chunk(2 * p, 0)
                k = 2 * p + 1

                @pl.when(k < nk)
                def _():
                    _chunk(k, 1)

                return 0

            lax.fori_loop(0, (nk + 1) // 2, body, 0)

            for b in range(2):
                @pl.when((nk - 1) % 2 == b)
                def _():
                    pltpu.make_async_copy(xt.at[_src(b)], rows_v[b],
                                          sem_g).wait()
                    pltpu.async_copy(rows_v[b], acc.at[dl_v[b]], sem_s,
                                     add=True)
                    pltpu.make_async_copy(rows_v[1 - b],
                                          acc.at[dl_v[1 - b]], sem_s).wait()
                    pltpu.make_async_copy(rows_v[b], acc.at[dl_v[b]],
                                          sem_s).wait()

        if kind == "ab":
            groups = ((((x1, p1),), outs[0]),
                      (((x0, p2),), outs[1]))
        else:
            groups = ((((x1, p1), (x0, p2)), outs[0]),)
        for arrays, out in groups:
            def _zrow(r, _):
                for j in range(_D // 16):
                    rows_v[0][r, pl.ds(j * 16, 16)] = jnp.zeros((16,), f32)
                return 0
            lax.fori_loop(0, _CH, _zrow, 0)
            nfull = q // _CH
            for k in range(nfull):
                pltpu.sync_copy(rows_v[0], acc.at[pl.ds(base + k * _CH, _CH)])
            rem = q - nfull * _CH
            if rem:
                pltpu.sync_copy(rows_v[0].at[pl.ds(0, rem)],
                                acc.at[pl.ds(base + nfull * _CH, rem)])
            plsc.subcore_barrier()
            for xt, pkt in arrays:
                _scan_edges(xt, pkt)
            plsc.subcore_barrier()
            ragged = (c == _NC - 1) & (s == _NT - 1)

            @pl.when(jnp.logical_not(ragged))
            def _():
                pltpu.sync_copy(acc.at[pl.ds(base, q)],
                                out.at[pl.ds(lo + base, q)])

            @pl.when(ragged)
            def _():
                pltpu.sync_copy(acc.at[pl.ds(base, last)],
                                out.at[pl.ds(lo + base, last)])

            plsc.subcore_barrier()

    return seg


_R = 1000


def _full(i):
    return (0, 0)


def _rowblk(i):
    return (i, 0)


@functools.lru_cache(maxsize=None)
def _build_tc_type1(n):
    grid = -(-n // _R)

    def body(x1, a, b_, gw1, gb1, gw2, gb2, hw, hb, out1, st1):
        i = pl.program_id(0)
        gin = x1[...] + a[...]
        t = jnp.maximum(gin @ gw1[...] + gb1[...], 0.0) @ gw2[...] + gb2[...]
        h1 = (t + (b_[...] @ hw[...] + hb[...]) * _COEF) * 0.5
        h1r = jnp.maximum(h1, 0.0)
        out1[...] = h1r

        @pl.when(i == 0)
        def _():
            st1[...] = jnp.zeros_like(st1)

        st1[0:1, :] += jnp.sum(h1r, axis=0, keepdims=True)
        st1[1:2, :] += jnp.sum(h1r * h1r, axis=0, keepdims=True)

    blk = pl.BlockSpec((_R, _D), _rowblk)
    wblk = pl.BlockSpec((_D, _D), _full)
    bblk = pl.BlockSpec((1, _D), _full)
    sblk = pl.BlockSpec((8, _D), _full)
    return pl.pallas_call(
        body,
        grid=(grid,),
        in_specs=[blk, blk, blk, wblk, bblk, wblk, bblk, wblk, bblk],
        out_specs=[blk, sblk],
        out_shape=[jax.ShapeDtypeStruct((n, _D), jnp.float32),
                   jax.ShapeDtypeStruct((8, _D), jnp.float32)],
    )


@functools.lru_cache(maxsize=None)
def _build_tc_type0(n):
    grid = -(-n // _R)

    def body(cacc, hw, hb, out0, st0):
        i = pl.program_id(0)
        h0 = (cacc[...] @ hw[...]) * (0.5 * _COEF) + hb[...] * _COEF
        h0r = jnp.maximum(h0, 0.0)
        out0[...] = h0r

        @pl.when(i == 0)
        def _():
            st0[...] = jnp.zeros_like(st0)

        st0[0:1, :] += jnp.sum(h0r, axis=0, keepdims=True)
        st0[1:2, :] += jnp.sum(h0r * h0r, axis=0, keepdims=True)

    blk = pl.BlockSpec((_R, _D), _rowblk)
    return pl.pallas_call(
        body,
        grid=(grid,),
        in_specs=[blk, pl.BlockSpec((_D, _D), _full),
                  pl.BlockSpec((1, _D), _full)],
        out_specs=[blk, pl.BlockSpec((8, _D), _full)],
        out_shape=[jax.ShapeDtypeStruct((n, _D), jnp.float32),
                   jax.ShapeDtypeStruct((8, _D), jnp.float32)],
    )


def _bn_apply(hr_blk, st, g, b, inv_n):
    m = st[0:1] * inv_n
    v = st[1:2] * inv_n - m * m
    scale = g * lax.rsqrt(v + _BN_EPS)
    return hr_blk * scale + (b - m * scale)


@functools.lru_cache(maxsize=None)
def _build_tc_norm2(n):
    grid = -(-n // _R)
    inv_n = 1.0 / n

    def body(h0r, h1r, st0, st1, g, b, o0, o1):
        gv, bv = g[...], b[...]
        o0[...] = _bn_apply(h0r[...], st0[...], gv, bv, inv_n)
        o1[...] = _bn_apply(h1r[...], st1[...], gv, bv, inv_n)

    blk = pl.BlockSpec((_R, _D), _rowblk)
    sblk = pl.BlockSpec((8, _D), _full)
    bblk = pl.BlockSpec((1, _D), _full)
    osd = jax.ShapeDtypeStruct((n, _D), jnp.float32)
    return pl.pallas_call(
        body,
        grid=(grid,),
        in_specs=[blk, blk, sblk, sblk, bblk, bblk],
        out_specs=[blk, blk],
        out_shape=[osd, osd],
    )


@functools.lru_cache(maxsize=None)
def _build_tc_norm_cat(n):
    nb = -(-n // _R)
    grid = 2 * nb
    inv_n = 1.0 / n

    def body(h0r, h1r, st0, st1, g, b, out):
        i = pl.program_id(0)
        gv, bv = g[...], b[...]
        y0 = _bn_apply(h0r[...], st0[...], gv, bv, inv_n)
        y1 = _bn_apply(h1r[...], st1[...], gv, bv, inv_n)
        out[...] = jnp.where(i < nb, y0, y1)

    blk0 = pl.BlockSpec((_R, _D), lambda i: (jnp.minimum(i, nb - 1), 0))
    blk1 = pl.BlockSpec((_R, _D), lambda i: (jnp.maximum(i - nb, 0), 0))
    sblk = pl.BlockSpec((8, _D), _full)
    bblk = pl.BlockSpec((1, _D), _full)
    return pl.pallas_call(
        body,
        grid=(grid,),
        in_specs=[blk0, blk1, sblk, sblk, bblk, bblk],
        out_specs=pl.BlockSpec((_R, _D), _rowblk),
        out_shape=jax.ShapeDtypeStruct((2 * n, _D), jnp.float32),
    )


_EPK = -(-_E // _CH) * _CH


def _layer(h0, h1, edges, gw1, gb1, gw2, gb2, hw, hb, bng, bnb, final):
    pk101, pk021, pk110, pk030 = edges
    r2 = lambda v: v.reshape(1, _D)
    a, b_ = _build_sc_segsum(_N, _EPK, "ab")(h0, h1, pk101, pk021)
    h1r, st1 = _build_tc_type1(_N)(h1, a, b_, gw1, r2(gb1), gw2, r2(gb2),
                                   hw, r2(hb))
    (cacc,) = _build_sc_segsum(_N, _EPK, "c")(h0, h1, pk110, pk030)
    h0r, st0 = _build_tc_type0(_N)(cacc, hw, r2(hb))
    if final:
        return _build_tc_norm_cat(_N)(h0r, h1r, st0, st1, r2(bng), r2(bnb))
    return _build_tc_norm2(_N)(h0r, h1r, st0, st1, r2(bng), r2(bnb))


def kernel(x0, x1, ei_101, ei_110, ei_021, ei_030,
           gin0_w1, gin0_b1, gin0_w2, gin0_b2, hl0_w, hl0_b, bn0_g, bn0_b,
           gin1_w1, gin1_b1, gin1_w2, gin1_b2, hl1_w, hl1_b, bn1_g, bn1_b):
    i32 = jnp.int32
    pad = _EPK - _E
    spad = jnp.zeros((pad,), i32)
    dpad = jnp.full((pad,), _N, i32)

    def _pack(ei):
        sp = jnp.concatenate([ei[0].astype(i32), spad]).reshape(-1, _CH)
        dp = jnp.concatenate([ei[1].astype(i32), dpad]).reshape(-1, _CH)
        return jnp.concatenate([sp, dp], axis=1).reshape(-1)

    edges = (_pack(ei_101), _pack(ei_021), _pack(ei_110), _pack(ei_030))
    h0, h1 = _layer(x0, x1, edges,
                    gin0_w1, gin0_b1, gin0_w2, gin0_b2, hl0_w, hl0_b,
                    bn0_g, bn0_b, final=False)
    return _layer(h0, h1, edges,
                  gin1_w1, gin1_b1, gin1_w2, gin1_b2, hl1_w, hl1_b,
                  bn1_g, bn1_b, final=True)
